# baseline (speedup 1.0000x reference)
import functools

import jax
import jax.numpy as jnp
from jax import lax
from jax.experimental import pallas as pl
from jax.experimental.pallas import tpu as pltpu
from jax.experimental.pallas import tpu_sc as plsc

N = 16384
D = 128
CLS = 10
CPAD = 16
W = 144
NC = 2
NS = 16
NW = NC * NS
ROWS_PER = N // NW
CHUNK = 128
NCHUNK = ROWS_PER // CHUNK
UNROLL = 4

_GDN = lax.GatherDimensionNumbers(
    offset_dims=(), collapsed_slice_dims=(0,), start_index_map=(0,))


def _bcast_last(x):
    idx = jnp.full((16, 1), 15, jnp.int32)
    return lax.gather(x, idx, _GDN, (1,),
                      mode=lax.GatherScatterMode.PROMISE_IN_BOUNDS)


def _sc_partials(xs, labels):
    mesh = plsc.VectorSubcoreMesh(core_axis_name="c", subcore_axis_name="s")

    @functools.partial(
        pl.kernel,
        out_type=jax.ShapeDtypeStruct((NW, CPAD * W), jnp.float32),
        mesh=mesh,
        compiler_params=pltpu.CompilerParams(needs_layout_passes=False),
        scratch_types=[
            pltpu.VMEM((CHUNK, D), jnp.float32),
            pltpu.VMEM((CHUNK, D), jnp.float32),
            pltpu.VMEM((CPAD * W,), jnp.float32),
            pltpu.VMEM((ROWS_PER,), jnp.int32),
            pltpu.SemaphoreType.DMA,
            pltpu.SemaphoreType.DMA,
        ],
    )
    def body(xs_hbm, lbl_hbm, out_hbm, buf_a, buf_b, tbl, lbl1d, sem_a, sem_b):
        cid = lax.axis_index("c")
        sid = lax.axis_index("s")
        wid = cid * NS + sid
        base = wid * ROWS_PER

        lane = lax.iota(jnp.int32, 16)
        zeros = jnp.zeros((16,), jnp.float32)

        for t in range(CPAD * W // 16):
            tbl[pl.ds(16 * t, 16)] = zeros

        pltpu.sync_copy(lbl_hbm.at[pl.ds(base, ROWS_PER)], lbl1d)

        col = [lane + 16 * j for j in range(9)]
        bufs = (buf_a, buf_b)
        sems = (sem_a, sem_b)

        def start(g):
            return pltpu.async_copy(
                xs_hbm.at[pl.ds(base + g * CHUNK, CHUNK)],
                bufs[g % 2], sems[g % 2])

        pending = start(0)

        def do_row(inbuf, goff, i):
            lblv = plsc.load_gather(lbl1d,
                                    [jnp.full((16,), goff, jnp.int32) + i])
            v = [inbuf[i, pl.ds(16 * j, 16)] for j in range(8)]
            sq01 = v[0] * v[0] + v[1] * v[1]
            sq23 = v[2] * v[2] + v[3] * v[3]
            sq45 = v[4] * v[4] + v[5] * v[5]
            sq67 = v[6] * v[6] + v[7] * v[7]
            sq = (sq01 + sq23) + (sq45 + sq67)
            sv = _bcast_last(plsc.cumsum(sq))
            ib = lax.bitcast_convert_type(sv, jnp.int32)
            y = lax.bitcast_convert_type(
                jnp.int32(0x5F3759DF) - (ib >> 1), jnp.float32)
            h = sv * jnp.float32(-0.5)
            y = y * (jnp.float32(1.5) + h * y * y)
            y = y * (jnp.float32(1.5) + h * y * y)
            y = jnp.minimum(y, jnp.float32(1e12))
            nsqv = sv * y * y
            aux = jnp.where(lane == 0, nsqv,
                            jnp.where(lane == 1, jnp.float32(1.0),
                                      jnp.float32(0.0)))
            addr = lblv * jnp.int32(W)
            for j in range(8):
                plsc.addupdate_scatter(tbl, [addr + col[j]], v[j] * y)
            plsc.addupdate_scatter(tbl, [addr + col[8]], aux)

        for g in range(NCHUNK):
            pending.wait()
            if g + 1 < NCHUNK:
                nxt = start(g + 1)
            inbuf = bufs[g % 2]
            goff = g * CHUNK

            @plsc.parallel_loop(0, CHUNK, step=1, unroll=UNROLL)
            def _rows(i, inbuf=inbuf, goff=goff):
                do_row(inbuf, goff, i)
            if g + 1 < NCHUNK:
                pending = nxt

        pltpu.sync_copy(tbl, out_hbm.at[wid])

    out3 = body(xs, labels)
    return out3


def _combine(part, center):
    def body(part_ref, center_ref, out_ref):
        p = jnp.sum(part_ref[...], axis=0)
        c = center_ref[...]
        S = p[:CLS, :D]
        dot = jnp.sum(S * c, axis=1, keepdims=True)
        cnsq = jnp.sum(c * c, axis=1, keepdims=True)
        A = p[:CLS, D:D + 1]
        cnt = p[:CLS, D + 1:D + 2]
        per = jnp.where(cnt > 0,
                        (A - 2.0 * dot) / jnp.maximum(cnt, 1.0) + cnsq,
                        0.0)
        out_ref[...] = jnp.sum(per).reshape(1, 1)

    return pl.pallas_call(
        body,
        out_shape=jax.ShapeDtypeStruct((1, 1), jnp.float32),
    )(part, center)


def kernel(xs, label, center):
    labels = label.astype(jnp.int32)
    part = _sc_partials(xs, labels)
    part = part.reshape(NW, CPAD, W)
    out = _combine(part, center)
    return out[0, 0]


# speedup vs baseline: 1.0274x; 1.0274x over previous
import functools

import jax
import jax.numpy as jnp
from jax import lax
from jax.experimental import pallas as pl
from jax.experimental.pallas import tpu as pltpu
from jax.experimental.pallas import tpu_sc as plsc

N = 16384
D = 128
CLS = 10
CPAD = 16
W = 144
NC = 2
NS = 16
NW = NC * NS
ROWS_PER = N // NW
CHUNK = 256
NCHUNK = ROWS_PER // CHUNK
UNROLL = 8

_GDN = lax.GatherDimensionNumbers(
    offset_dims=(), collapsed_slice_dims=(0,), start_index_map=(0,))


def _bcast_last(x):
    idx = jnp.full((16, 1), 15, jnp.int32)
    return lax.gather(x, idx, _GDN, (1,),
                      mode=lax.GatherScatterMode.PROMISE_IN_BOUNDS)


def _sc_partials(xs, labels):
    mesh = plsc.VectorSubcoreMesh(core_axis_name="c", subcore_axis_name="s")

    @functools.partial(
        pl.kernel,
        out_type=jax.ShapeDtypeStruct((NW, CPAD * W), jnp.float32),
        mesh=mesh,
        compiler_params=pltpu.CompilerParams(needs_layout_passes=False),
        scratch_types=[
            pltpu.VMEM((CHUNK, D), jnp.float32),
            pltpu.VMEM((CHUNK, D), jnp.float32),
            pltpu.VMEM((CPAD * W,), jnp.float32),
            pltpu.VMEM((ROWS_PER,), jnp.int32),
            pltpu.SemaphoreType.DMA,
            pltpu.SemaphoreType.DMA,
        ],
    )
    def body(xs_hbm, lbl_hbm, out_hbm, buf_a, buf_b, tbl, lbl1d, sem_a, sem_b):
        cid = lax.axis_index("c")
        sid = lax.axis_index("s")
        wid = cid * NS + sid
        base = wid * ROWS_PER

        lane = lax.iota(jnp.int32, 16)
        zeros = jnp.zeros((16,), jnp.float32)

        for t in range(CPAD * W // 16):
            tbl[pl.ds(16 * t, 16)] = zeros

        pltpu.sync_copy(lbl_hbm.at[pl.ds(base, ROWS_PER)], lbl1d)

        col = [lane + 16 * j for j in range(9)]
        bufs = (buf_a, buf_b)
        sems = (sem_a, sem_b)

        def start(g):
            return pltpu.async_copy(
                xs_hbm.at[pl.ds(base + g * CHUNK, CHUNK)],
                bufs[g % 2], sems[g % 2])

        pending = start(0)

        def do_row(inbuf, goff, i):
            lblv = plsc.load_gather(lbl1d,
                                    [jnp.full((16,), goff, jnp.int32) + i])
            v = [inbuf[i, pl.ds(16 * j, 16)] for j in range(8)]
            sq01 = v[0] * v[0] + v[1] * v[1]
            sq23 = v[2] * v[2] + v[3] * v[3]
            sq45 = v[4] * v[4] + v[5] * v[5]
            sq67 = v[6] * v[6] + v[7] * v[7]
            sq = (sq01 + sq23) + (sq45 + sq67)
            sv = _bcast_last(plsc.cumsum(sq))
            ib = lax.bitcast_convert_type(sv, jnp.int32)
            y = lax.bitcast_convert_type(
                jnp.int32(0x5F3759DF) - (ib >> 1), jnp.float32)
            h = sv * jnp.float32(-0.5)
            y = y * (jnp.float32(1.5) + h * y * y)
            y = y * (jnp.float32(1.5) + h * y * y)
            y = jnp.minimum(y, jnp.float32(1e12))
            nsqv = sv * y * y
            aux = jnp.where(lane == 0, nsqv,
                            jnp.where(lane == 1, jnp.float32(1.0),
                                      jnp.float32(0.0)))
            addr = lblv * jnp.int32(W)
            for j in range(8):
                plsc.addupdate_scatter(tbl, [addr + col[j]], v[j] * y)
            plsc.addupdate_scatter(tbl, [addr + col[8]], aux)

        for g in range(NCHUNK):
            pending.wait()
            if g + 1 < NCHUNK:
                nxt = start(g + 1)
            inbuf = bufs[g % 2]
            goff = g * CHUNK

            @plsc.parallel_loop(0, CHUNK, step=1, unroll=UNROLL)
            def _rows(i, inbuf=inbuf, goff=goff):
                do_row(inbuf, goff, i)
            if g + 1 < NCHUNK:
                pending = nxt

        pltpu.sync_copy(tbl, out_hbm.at[wid])

    out3 = body(xs, labels)
    return out3


def _combine(part, center):
    def body(part_ref, center_ref, out_ref):
        p = jnp.sum(part_ref[...], axis=0)
        c = center_ref[...]
        S = p[:CLS, :D]
        dot = jnp.sum(S * c, axis=1, keepdims=True)
        cnsq = jnp.sum(c * c, axis=1, keepdims=True)
        A = p[:CLS, D:D + 1]
        cnt = p[:CLS, D + 1:D + 2]
        per = jnp.where(cnt > 0,
                        (A - 2.0 * dot) / jnp.maximum(cnt, 1.0) + cnsq,
                        0.0)
        out_ref[...] = jnp.sum(per).reshape(1, 1)

    return pl.pallas_call(
        body,
        out_shape=jax.ShapeDtypeStruct((1, 1), jnp.float32),
    )(part, center)


def kernel(xs, label, center):
    labels = label.astype(jnp.int32)
    part = _sc_partials(xs, labels)
    part = part.reshape(NW, CPAD, W)
    out = _combine(part, center)
    return out[0, 0]


# speedup vs baseline: 1.0863x; 1.0573x over previous
import functools

import jax
import jax.numpy as jnp
from jax import lax
from jax.experimental import pallas as pl
from jax.experimental.pallas import tpu as pltpu
from jax.experimental.pallas import tpu_sc as plsc

N = 16384
D = 128
CLS = 10
CPAD = 16
W = 256
NC = 2
NS = 16
NW = NC * NS
ROWS_PER = N // NW
CHUNK = 256
NCHUNK = ROWS_PER // CHUNK
UNROLL = 8

_GDN = lax.GatherDimensionNumbers(
    offset_dims=(), collapsed_slice_dims=(0,), start_index_map=(0,))


def _bcast_last(x):
    idx = jnp.full((16, 1), 15, jnp.int32)
    return lax.gather(x, idx, _GDN, (1,),
                      mode=lax.GatherScatterMode.PROMISE_IN_BOUNDS)


def _sc_partials(xs, labels):
    mesh = plsc.VectorSubcoreMesh(core_axis_name="c", subcore_axis_name="s")

    @functools.partial(
        pl.kernel,
        out_type=jax.ShapeDtypeStruct((NW, CPAD * W), jnp.float32),
        mesh=mesh,
        compiler_params=pltpu.CompilerParams(needs_layout_passes=False),
        scratch_types=[
            pltpu.VMEM((CHUNK, D), jnp.float32),
            pltpu.VMEM((CHUNK, D), jnp.float32),
            pltpu.VMEM((CPAD * W,), jnp.float32),
            pltpu.VMEM((ROWS_PER,), jnp.int32),
            pltpu.SemaphoreType.DMA,
            pltpu.SemaphoreType.DMA,
        ],
    )
    def body(xs_hbm, lbl_hbm, out_hbm, buf_a, buf_b, tbl, lbl1d, sem_a, sem_b):
        cid = lax.axis_index("c")
        sid = lax.axis_index("s")
        wid = cid * NS + sid
        base = wid * ROWS_PER

        lane = lax.iota(jnp.int32, 16)
        zeros = jnp.zeros((16,), jnp.float32)

        for t in range(CPAD * W // 16):
            tbl[pl.ds(16 * t, 16)] = zeros

        pltpu.sync_copy(lbl_hbm.at[pl.ds(base, ROWS_PER)], lbl1d)

        col = [lane + 16 * j for j in range(9)]
        bufs = (buf_a, buf_b)
        sems = (sem_a, sem_b)

        def start(g):
            return pltpu.async_copy(
                xs_hbm.at[pl.ds(base + g * CHUNK, CHUNK)],
                bufs[g % 2], sems[g % 2])

        pending = start(0)

        def do_row(inbuf, goff, i):
            lblv = plsc.load_gather(lbl1d,
                                    [jnp.full((16,), goff, jnp.int32) + i])
            v = [inbuf[i, pl.ds(16 * j, 16)] for j in range(8)]
            sq01 = v[0] * v[0] + v[1] * v[1]
            sq23 = v[2] * v[2] + v[3] * v[3]
            sq45 = v[4] * v[4] + v[5] * v[5]
            sq67 = v[6] * v[6] + v[7] * v[7]
            sq = (sq01 + sq23) + (sq45 + sq67)
            sv = _bcast_last(plsc.cumsum(sq))
            ib = lax.bitcast_convert_type(sv, jnp.int32)
            y = lax.bitcast_convert_type(
                jnp.int32(0x5F3759DF) - (ib >> 1), jnp.float32)
            h = sv * jnp.float32(-0.5)
            y = y * (jnp.float32(1.5) + h * y * y)
            y = y * (jnp.float32(1.5) + h * y * y)
            y = jnp.minimum(y, jnp.float32(1e12))
            nsqv = sv * y * y
            aux = jnp.where(lane == 0, nsqv,
                            jnp.where(lane == 1, jnp.float32(1.0),
                                      jnp.float32(0.0)))
            addr = lblv * jnp.int32(W)
            for j in range(8):
                plsc.addupdate_scatter(tbl, [addr + col[j]], v[j] * y)
            plsc.addupdate_scatter(tbl, [addr + col[8]], aux)

        for g in range(NCHUNK):
            pending.wait()
            if g + 1 < NCHUNK:
                nxt = start(g + 1)
            inbuf = bufs[g % 2]
            goff = g * CHUNK

            @plsc.parallel_loop(0, CHUNK, step=1, unroll=UNROLL)
            def _rows(i, inbuf=inbuf, goff=goff):
                do_row(inbuf, goff, i)
            if g + 1 < NCHUNK:
                pending = nxt

        pltpu.sync_copy(tbl, out_hbm.at[wid])

    out3 = body(xs, labels)
    return out3


def _combine(part, center):
    def body(part_ref, center_ref, out_ref):
        p3 = part_ref[...].reshape(NW, CPAD, W)
        p = jnp.sum(p3, axis=0)
        c = center_ref[...]
        S = p[:CLS, :D]
        dot = jnp.sum(S * c, axis=1, keepdims=True)
        cnsq = jnp.sum(c * c, axis=1, keepdims=True)
        A = p[:CLS, D:D + 1]
        cnt = p[:CLS, D + 1:D + 2]
        per = jnp.where(cnt > 0,
                        (A - 2.0 * dot) / jnp.maximum(cnt, 1.0) + cnsq,
                        0.0)
        out_ref[...] = jnp.sum(per).reshape(1, 1)

    return pl.pallas_call(
        body,
        out_shape=jax.ShapeDtypeStruct((1, 1), jnp.float32),
    )(part, center)


def kernel(xs, label, center):
    labels = label.astype(jnp.int32)
    part = _sc_partials(xs, labels)
    out = _combine(part, center)
    return out[0, 0]


# speedup vs baseline: 1.1299x; 1.0402x over previous
import functools

import jax
import jax.numpy as jnp
from jax import lax
from jax.experimental import pallas as pl
from jax.experimental.pallas import tpu as pltpu
from jax.experimental.pallas import tpu_sc as plsc

N = 16384
D = 128
CLS = 10
CPAD = 16
W = 256
NC = 2
NS = 16
NW = NC * NS

N_SC = 8192
ROWS_PER = N_SC // NW
CHUNK = 256
NCHUNK = ROWS_PER // CHUNK
UNROLL = 8

TC_BLK = 512
NBLK = N // TC_BLK
BLK0_TC = N_SC // TC_BLK

_GDN = lax.GatherDimensionNumbers(
    offset_dims=(), collapsed_slice_dims=(0,), start_index_map=(0,))


def _bcast_last(x):
    idx = jnp.full((16, 1), 15, jnp.int32)
    return lax.gather(x, idx, _GDN, (1,),
                      mode=lax.GatherScatterMode.PROMISE_IN_BOUNDS)


def _sc_partials(xs, labels):
    mesh = plsc.VectorSubcoreMesh(core_axis_name="c", subcore_axis_name="s")

    @functools.partial(
        pl.kernel,
        out_type=jax.ShapeDtypeStruct((NW, CPAD * W), jnp.float32),
        mesh=mesh,
        compiler_params=pltpu.CompilerParams(needs_layout_passes=False),
        scratch_types=[
            pltpu.VMEM((CHUNK, D), jnp.float32),
            pltpu.VMEM((CPAD * W,), jnp.float32),
            pltpu.VMEM((ROWS_PER,), jnp.int32),
            pltpu.SemaphoreType.DMA,
        ],
    )
    def body(xs_hbm, lbl_hbm, out_hbm, inbuf, tbl, lbl1d, sem):
        cid = lax.axis_index("c")
        sid = lax.axis_index("s")
        wid = cid * NS + sid
        base = wid * ROWS_PER

        lane = lax.iota(jnp.int32, 16)
        zeros = jnp.zeros((16,), jnp.float32)

        pending = pltpu.async_copy(
            xs_hbm.at[pl.ds(base, CHUNK)], inbuf, sem)

        for t in range(CPAD * W // 16):
            tbl[pl.ds(16 * t, 16)] = zeros

        pltpu.sync_copy(lbl_hbm.at[pl.ds(base, ROWS_PER)], lbl1d)

        col = [lane + 16 * j for j in range(9)]

        pending.wait()

        @plsc.parallel_loop(0, CHUNK, step=1, unroll=UNROLL)
        def _rows(i):
            lblv = plsc.load_gather(lbl1d, [jnp.full((16,), i, jnp.int32)])
            v = [inbuf[i, pl.ds(16 * j, 16)] for j in range(8)]
            sq01 = v[0] * v[0] + v[1] * v[1]
            sq23 = v[2] * v[2] + v[3] * v[3]
            sq45 = v[4] * v[4] + v[5] * v[5]
            sq67 = v[6] * v[6] + v[7] * v[7]
            sq = (sq01 + sq23) + (sq45 + sq67)
            sv = _bcast_last(plsc.cumsum(sq))
            ib = lax.bitcast_convert_type(sv, jnp.int32)
            y = lax.bitcast_convert_type(
                jnp.int32(0x5F3759DF) - (ib >> 1), jnp.float32)
            h = sv * jnp.float32(-0.5)
            y = y * (jnp.float32(1.5) + h * y * y)
            y = y * (jnp.float32(1.5) + h * y * y)
            y = jnp.minimum(y, jnp.float32(1e12))
            nsqv = sv * y * y
            aux = jnp.where(lane == 0, nsqv,
                            jnp.where(lane == 1, jnp.float32(1.0),
                                      jnp.float32(0.0)))
            addr = lblv * jnp.int32(W)
            for j in range(8):
                plsc.addupdate_scatter(tbl, [addr + col[j]], v[j] * y)
            plsc.addupdate_scatter(tbl, [addr + col[8]], aux)

        pltpu.sync_copy(tbl, out_hbm.at[wid])

    return body(xs, labels)


def _tc_partials(xs, labels3):
    def body(x_ref, lbl_ref, out_ref):
        g = pl.program_id(0)
        x = x_ref[...]
        nrm2 = jnp.sum(x * x, axis=1, keepdims=True)
        scale = jnp.minimum(lax.rsqrt(nrm2), jnp.float32(1e12))
        xh = x * scale
        nsq = nrm2 * scale * scale
        lbl = lbl_ref[0, 0]
        onehot = (lbl[:, None] ==
                  lax.broadcasted_iota(jnp.int32, (1, CPAD), 1)
                  ).astype(jnp.float32)
        aux = jnp.concatenate(
            [nsq, jnp.ones((TC_BLK, 1), jnp.float32),
             jnp.zeros((TC_BLK, W - D - 2), jnp.float32)], axis=1)
        rhs = jnp.concatenate([xh, aux], axis=1)
        part = jax.lax.dot_general(
            onehot, rhs, (((0,), (0,)), ((), ())),
            preferred_element_type=jnp.float32)

        @pl.when(g == 0)
        def _init():
            out_ref[...] = jnp.zeros((CPAD, W), jnp.float32)

        out_ref[...] += part

    return pl.pallas_call(
        body,
        grid=(NBLK - BLK0_TC,),
        in_specs=[
            pl.BlockSpec((TC_BLK, D), lambda g: (g + BLK0_TC, 0)),
            pl.BlockSpec((1, 1, TC_BLK), lambda g: (g + BLK0_TC, 0, 0)),
        ],
        out_specs=pl.BlockSpec((CPAD, W), lambda g: (0, 0)),
        out_shape=jax.ShapeDtypeStruct((CPAD, W), jnp.float32),
    )(xs, labels3)


def _combine(part_sc, part_tc, center):
    def body(psc_ref, ptc_ref, center_ref, out_ref):
        p3 = psc_ref[...].reshape(NW, CPAD, W)
        p = jnp.sum(p3, axis=0) + ptc_ref[...]
        c = center_ref[...]
        S = p[:CLS, :D]
        dot = jnp.sum(S * c, axis=1, keepdims=True)
        cnsq = jnp.sum(c * c, axis=1, keepdims=True)
        A = p[:CLS, D:D + 1]
        cnt = p[:CLS, D + 1:D + 2]
        per = jnp.where(cnt > 0,
                        (A - 2.0 * dot) / jnp.maximum(cnt, 1.0) + cnsq,
                        0.0)
        out_ref[...] = jnp.sum(per).reshape(1, 1)

    return pl.pallas_call(
        body,
        out_shape=jax.ShapeDtypeStruct((1, 1), jnp.float32),
    )(part_sc, part_tc, center)


def kernel(xs, label, center):
    labels = label.astype(jnp.int32)
    part_sc = _sc_partials(xs, labels)
    part_tc = _tc_partials(xs, labels.reshape(NBLK, 1, TC_BLK))
    out = _combine(part_sc, part_tc, center)
    return out[0, 0]


# speedup vs baseline: 1.2657x; 1.1201x over previous
import functools

import jax
import jax.numpy as jnp
from jax import lax
from jax.experimental import pallas as pl
from jax.experimental.pallas import tpu as pltpu
from jax.experimental.pallas import tpu_sc as plsc

N = 16384
D = 128
CLS = 10
CPAD = 16
W = 256
NC = 2
NS = 16
NW = NC * NS

N_SC = 8192
ROWS_PER = N_SC // NW
CHUNK = 256
NCHUNK = ROWS_PER // CHUNK
UNROLL = 8

TC_BLK = 2048
NBLK = N // TC_BLK
BLK0_TC = N_SC // TC_BLK

_GDN = lax.GatherDimensionNumbers(
    offset_dims=(), collapsed_slice_dims=(0,), start_index_map=(0,))


def _bcast_last(x):
    idx = jnp.full((16, 1), 15, jnp.int32)
    return lax.gather(x, idx, _GDN, (1,),
                      mode=lax.GatherScatterMode.PROMISE_IN_BOUNDS)


def _sc_partials(xs, labels):
    mesh = plsc.VectorSubcoreMesh(core_axis_name="c", subcore_axis_name="s")

    @functools.partial(
        pl.kernel,
        out_type=jax.ShapeDtypeStruct((NW, CPAD * W), jnp.float32),
        mesh=mesh,
        compiler_params=pltpu.CompilerParams(needs_layout_passes=False),
        scratch_types=[
            pltpu.VMEM((CHUNK, D), jnp.float32),
            pltpu.VMEM((CPAD * W,), jnp.float32),
            pltpu.VMEM((ROWS_PER,), jnp.int32),
            pltpu.SemaphoreType.DMA,
        ],
    )
    def body(xs_hbm, lbl_hbm, out_hbm, inbuf, tbl, lbl1d, sem):
        cid = lax.axis_index("c")
        sid = lax.axis_index("s")
        wid = cid * NS + sid
        base = wid * ROWS_PER

        lane = lax.iota(jnp.int32, 16)
        zeros = jnp.zeros((16,), jnp.float32)

        pending = pltpu.async_copy(
            xs_hbm.at[pl.ds(base, CHUNK)], inbuf, sem)

        for t in range(CPAD * W // 16):
            tbl[pl.ds(16 * t, 16)] = zeros

        pltpu.sync_copy(lbl_hbm.at[pl.ds(base, ROWS_PER)], lbl1d)

        col = [lane + 16 * j for j in range(9)]

        pending.wait()

        @plsc.parallel_loop(0, CHUNK, step=1, unroll=UNROLL)
        def _rows(i):
            lblv = plsc.load_gather(lbl1d, [jnp.full((16,), i, jnp.int32)])
            v = [inbuf[i, pl.ds(16 * j, 16)] for j in range(8)]
            sq01 = v[0] * v[0] + v[1] * v[1]
            sq23 = v[2] * v[2] + v[3] * v[3]
            sq45 = v[4] * v[4] + v[5] * v[5]
            sq67 = v[6] * v[6] + v[7] * v[7]
            sq = (sq01 + sq23) + (sq45 + sq67)
            sv = _bcast_last(plsc.cumsum(sq))
            ib = lax.bitcast_convert_type(sv, jnp.int32)
            y = lax.bitcast_convert_type(
                jnp.int32(0x5F3759DF) - (ib >> 1), jnp.float32)
            h = sv * jnp.float32(-0.5)
            y = y * (jnp.float32(1.5) + h * y * y)
            y = y * (jnp.float32(1.5) + h * y * y)
            y = jnp.minimum(y, jnp.float32(1e12))
            nsqv = sv * y * y
            aux = jnp.where(lane == 0, nsqv,
                            jnp.where(lane == 1, jnp.float32(1.0),
                                      jnp.float32(0.0)))
            addr = lblv * jnp.int32(W)
            for j in range(8):
                plsc.addupdate_scatter(tbl, [addr + col[j]], v[j] * y)
            plsc.addupdate_scatter(tbl, [addr + col[8]], aux)

        pltpu.sync_copy(tbl, out_hbm.at[wid])

    return body(xs, labels)


def _tc_partials(xs, labels3):
    def body(x_ref, lbl_ref, out_ref):
        g = pl.program_id(0)
        x = x_ref[...]
        nrm2 = jnp.sum(x * x, axis=1, keepdims=True)
        scale = jnp.minimum(lax.rsqrt(nrm2), jnp.float32(1e12))
        xh = x * scale
        nsq = nrm2 * scale * scale
        lbl = lbl_ref[0, 0]
        onehot_t = (lax.broadcasted_iota(jnp.int32, (CPAD, TC_BLK), 0) ==
                    lbl[None, :]).astype(jnp.float32)
        s_tbl = jax.lax.dot_general(
            onehot_t, xh, (((1,), (0,)), ((), ())),
            preferred_element_type=jnp.float32)
        a_tbl = jax.lax.dot_general(
            onehot_t, nsq, (((1,), (0,)), ((), ())),
            preferred_element_type=jnp.float32)
        cnt = jnp.sum(onehot_t, axis=1, keepdims=True)
        colidx = lax.broadcasted_iota(jnp.int32, (CPAD, W - D), 1)
        second = jnp.where(colidx == 0, a_tbl,
                           jnp.where(colidx == 1, cnt, jnp.float32(0.0)))
        part = jnp.concatenate([s_tbl, second], axis=1)

        @pl.when(g == 0)
        def _init():
            out_ref[...] = jnp.zeros((CPAD, W), jnp.float32)

        out_ref[...] += part

    return pl.pallas_call(
        body,
        grid=(NBLK - BLK0_TC,),
        in_specs=[
            pl.BlockSpec((TC_BLK, D), lambda g: (g + BLK0_TC, 0)),
            pl.BlockSpec((1, 1, TC_BLK), lambda g: (g + BLK0_TC, 0, 0)),
        ],
        out_specs=pl.BlockSpec((CPAD, W), lambda g: (0, 0)),
        out_shape=jax.ShapeDtypeStruct((CPAD, W), jnp.float32),
    )(xs, labels3)


def _combine(part_sc, part_tc, center):
    def body(psc_ref, ptc_ref, center_ref, out_ref):
        p3 = psc_ref[...].reshape(NW, CPAD, W)
        p = jnp.sum(p3, axis=0) + ptc_ref[...]
        c = center_ref[...]
        S = p[:CLS, :D]
        dot = jnp.sum(S * c, axis=1, keepdims=True)
        cnsq = jnp.sum(c * c, axis=1, keepdims=True)
        A = p[:CLS, D:D + 1]
        cnt = p[:CLS, D + 1:D + 2]
        per = jnp.where(cnt > 0,
                        (A - 2.0 * dot) / jnp.maximum(cnt, 1.0) + cnsq,
                        0.0)
        out_ref[...] = jnp.sum(per).reshape(1, 1)

    return pl.pallas_call(
        body,
        out_shape=jax.ShapeDtypeStruct((1, 1), jnp.float32),
    )(part_sc, part_tc, center)


def kernel(xs, label, center):
    labels = label.astype(jnp.int32)
    part_sc = _sc_partials(xs, labels)
    part_tc = _tc_partials(xs, labels.reshape(N // TC_BLK, 1, TC_BLK))
    out = _combine(part_sc, part_tc, center)
    return out[0, 0]


# speedup vs baseline: 1.3277x; 1.0490x over previous
import functools

import jax
import jax.numpy as jnp
from jax import lax
from jax.experimental import pallas as pl
from jax.experimental.pallas import tpu as pltpu
from jax.experimental.pallas import tpu_sc as plsc

N = 16384
D = 128
CLS = 10
CPAD = 16
W = 256
NC = 2
NS = 16
NW = NC * NS

N_SC = 6144
ROWS_PER = N_SC // NW
NCHUNK = 2
CHUNK = ROWS_PER // NCHUNK
UNROLL = 8

TC_BLK = 2048
NBLK = N // TC_BLK
BLK0_TC = N_SC // TC_BLK

_GDN = lax.GatherDimensionNumbers(
    offset_dims=(), collapsed_slice_dims=(0,), start_index_map=(0,))


def _bcast_last(x):
    idx = jnp.full((16, 1), 15, jnp.int32)
    return lax.gather(x, idx, _GDN, (1,),
                      mode=lax.GatherScatterMode.PROMISE_IN_BOUNDS)


def _sc_partials(xs, labels):
    mesh = plsc.VectorSubcoreMesh(core_axis_name="c", subcore_axis_name="s")

    @functools.partial(
        pl.kernel,
        out_type=jax.ShapeDtypeStruct((NW, CPAD * W), jnp.float32),
        mesh=mesh,
        compiler_params=pltpu.CompilerParams(needs_layout_passes=False),
        scratch_types=[
            pltpu.VMEM((CHUNK, D), jnp.float32),
            pltpu.VMEM((CHUNK, D), jnp.float32),
            pltpu.VMEM((CPAD * W,), jnp.float32),
            pltpu.VMEM((ROWS_PER,), jnp.int32),
            pltpu.SemaphoreType.DMA,
            pltpu.SemaphoreType.DMA,
        ],
    )
    def body(xs_hbm, lbl_hbm, out_hbm, buf_a, buf_b, tbl, lbl1d,
             sem_a, sem_b):
        cid = lax.axis_index("c")
        sid = lax.axis_index("s")
        wid = cid * NS + sid
        base = wid * ROWS_PER

        lane = lax.iota(jnp.int32, 16)
        zeros = jnp.zeros((16,), jnp.float32)

        bufs = (buf_a, buf_b)
        sems = (sem_a, sem_b)

        def start(g):
            return pltpu.async_copy(
                xs_hbm.at[pl.ds(base + g * CHUNK, CHUNK)],
                bufs[g % 2], sems[g % 2])

        pending = start(0)

        for t in range(CPAD * W // 16):
            tbl[pl.ds(16 * t, 16)] = zeros

        pltpu.sync_copy(lbl_hbm.at[pl.ds(base, ROWS_PER)], lbl1d)

        col = [lane + 16 * j for j in range(9)]

        for g in range(NCHUNK):
            pending.wait()
            if g + 1 < NCHUNK:
                nxt = start(g + 1)
            inbuf = bufs[g % 2]
            goff = g * CHUNK

            @plsc.parallel_loop(0, CHUNK, step=1, unroll=UNROLL)
            def _rows(i, inbuf=inbuf, goff=goff):
                lblv = plsc.load_gather(
                    lbl1d, [jnp.full((16,), goff, jnp.int32) + i])
                v = [inbuf[i, pl.ds(16 * j, 16)] for j in range(8)]
                sq01 = v[0] * v[0] + v[1] * v[1]
                sq23 = v[2] * v[2] + v[3] * v[3]
                sq45 = v[4] * v[4] + v[5] * v[5]
                sq67 = v[6] * v[6] + v[7] * v[7]
                sq = (sq01 + sq23) + (sq45 + sq67)
                sv = _bcast_last(plsc.cumsum(sq))
                ib = lax.bitcast_convert_type(sv, jnp.int32)
                y = lax.bitcast_convert_type(
                    jnp.int32(0x5F3759DF) - (ib >> 1), jnp.float32)
                h = sv * jnp.float32(-0.5)
                y = y * (jnp.float32(1.5) + h * y * y)
                y = y * (jnp.float32(1.5) + h * y * y)
                y = jnp.minimum(y, jnp.float32(1e12))
                nsqv = sv * y * y
                aux = jnp.where(lane == 0, nsqv,
                                jnp.where(lane == 1, jnp.float32(1.0),
                                          jnp.float32(0.0)))
                addr = lblv * jnp.int32(W)
                for j in range(8):
                    plsc.addupdate_scatter(tbl, [addr + col[j]], v[j] * y)
                plsc.addupdate_scatter(tbl, [addr + col[8]], aux)

            if g + 1 < NCHUNK:
                pending = nxt

        pltpu.sync_copy(tbl, out_hbm.at[wid])

    return body(xs, labels)


def _tc_partials(xs, labels3):
    def body(x_ref, lbl_ref, out_ref):
        g = pl.program_id(0)
        x = x_ref[...]
        nrm2 = jnp.sum(x * x, axis=1, keepdims=True)
        scale = jnp.minimum(lax.rsqrt(nrm2), jnp.float32(1e12))
        xh = x * scale
        nsq = nrm2 * scale * scale
        lbl = lbl_ref[0, 0]
        onehot_t = (lax.broadcasted_iota(jnp.int32, (CPAD, TC_BLK), 0) ==
                    lbl[None, :]).astype(jnp.float32)
        s_tbl = jax.lax.dot_general(
            onehot_t, xh, (((1,), (0,)), ((), ())),
            preferred_element_type=jnp.float32)
        a_tbl = jax.lax.dot_general(
            onehot_t, nsq, (((1,), (0,)), ((), ())),
            preferred_element_type=jnp.float32)
        cnt = jnp.sum(onehot_t, axis=1, keepdims=True)
        colidx = lax.broadcasted_iota(jnp.int32, (CPAD, W - D), 1)
        second = jnp.where(colidx == 0, a_tbl,
                           jnp.where(colidx == 1, cnt, jnp.float32(0.0)))
        part = jnp.concatenate([s_tbl, second], axis=1)

        @pl.when(g == 0)
        def _init():
            out_ref[...] = jnp.zeros((CPAD, W), jnp.float32)

        out_ref[...] += part

    return pl.pallas_call(
        body,
        grid=(NBLK - BLK0_TC,),
        in_specs=[
            pl.BlockSpec((TC_BLK, D), lambda g: (g + BLK0_TC, 0)),
            pl.BlockSpec((1, 1, TC_BLK), lambda g: (g + BLK0_TC, 0, 0)),
        ],
        out_specs=pl.BlockSpec((CPAD, W), lambda g: (0, 0)),
        out_shape=jax.ShapeDtypeStruct((CPAD, W), jnp.float32),
    )(xs, labels3)


def _combine(part_sc, part_tc, center):
    def body(psc_ref, ptc_ref, center_ref, out_ref):
        p3 = psc_ref[...].reshape(NW, CPAD, W)
        p = jnp.sum(p3, axis=0) + ptc_ref[...]
        c = center_ref[...]
        S = p[:CLS, :D]
        dot = jnp.sum(S * c, axis=1, keepdims=True)
        cnsq = jnp.sum(c * c, axis=1, keepdims=True)
        A = p[:CLS, D:D + 1]
        cnt = p[:CLS, D + 1:D + 2]
        per = jnp.where(cnt > 0,
                        (A - 2.0 * dot) / jnp.maximum(cnt, 1.0) + cnsq,
                        0.0)
        out_ref[...] = jnp.sum(per).reshape(1, 1)

    return pl.pallas_call(
        body,
        out_shape=jax.ShapeDtypeStruct((1, 1), jnp.float32),
    )(part_sc, part_tc, center)


def kernel(xs, label, center):
    labels = label.astype(jnp.int32)
    part_sc = _sc_partials(xs, labels)
    part_tc = _tc_partials(xs, labels.reshape(N // TC_BLK, 1, TC_BLK))
    out = _combine(part_sc, part_tc, center)
    return out[0, 0]


# speedup vs baseline: 1.3744x; 1.0352x over previous
import functools

import jax
import jax.numpy as jnp
from jax import lax
from jax.experimental import pallas as pl
from jax.experimental.pallas import tpu as pltpu
from jax.experimental.pallas import tpu_sc as plsc

N = 16384
D = 128
CLS = 10
CPAD = 16
W = 256
NC = 2
NS = 16
NW = NC * NS

N_SC = 4096
ROWS_PER = N_SC // NW
NCHUNK = 2
CHUNK = ROWS_PER // NCHUNK
UNROLL = 8

TC_BLK = 2048
NBLK = N // TC_BLK
BLK0_TC = N_SC // TC_BLK

_GDN = lax.GatherDimensionNumbers(
    offset_dims=(), collapsed_slice_dims=(0,), start_index_map=(0,))


def _bcast_last(x):
    idx = jnp.full((16, 1), 15, jnp.int32)
    return lax.gather(x, idx, _GDN, (1,),
                      mode=lax.GatherScatterMode.PROMISE_IN_BOUNDS)


def _sc_partials(xs, labels):
    mesh = plsc.VectorSubcoreMesh(core_axis_name="c", subcore_axis_name="s")

    @functools.partial(
        pl.kernel,
        out_type=jax.ShapeDtypeStruct((NW, CPAD * W), jnp.float32),
        mesh=mesh,
        compiler_params=pltpu.CompilerParams(needs_layout_passes=False),
        scratch_types=[
            pltpu.VMEM((CHUNK, D), jnp.float32),
            pltpu.VMEM((CHUNK, D), jnp.float32),
            pltpu.VMEM((CPAD * W,), jnp.float32),
            pltpu.VMEM((ROWS_PER,), jnp.int32),
            pltpu.SemaphoreType.DMA,
            pltpu.SemaphoreType.DMA,
        ],
    )
    def body(xs_hbm, lbl_hbm, out_hbm, buf_a, buf_b, tbl, lbl1d,
             sem_a, sem_b):
        cid = lax.axis_index("c")
        sid = lax.axis_index("s")
        wid = cid * NS + sid
        base = wid * ROWS_PER

        lane = lax.iota(jnp.int32, 16)
        zeros = jnp.zeros((16,), jnp.float32)

        bufs = (buf_a, buf_b)
        sems = (sem_a, sem_b)

        def start(g):
            return pltpu.async_copy(
                xs_hbm.at[pl.ds(base + g * CHUNK, CHUNK)],
                bufs[g % 2], sems[g % 2])

        pending = start(0)

        for t in range(CPAD * W // 16):
            tbl[pl.ds(16 * t, 16)] = zeros

        pltpu.sync_copy(lbl_hbm.at[pl.ds(base, ROWS_PER)], lbl1d)

        col = [lane + 16 * j for j in range(9)]

        for g in range(NCHUNK):
            pending.wait()
            if g + 1 < NCHUNK:
                nxt = start(g + 1)
            inbuf = bufs[g % 2]
            goff = g * CHUNK

            @plsc.parallel_loop(0, CHUNK, step=1, unroll=UNROLL)
            def _rows(i, inbuf=inbuf, goff=goff):
                lblv = plsc.load_gather(
                    lbl1d, [jnp.full((16,), goff, jnp.int32) + i])
                v = [inbuf[i, pl.ds(16 * j, 16)] for j in range(8)]
                sq01 = v[0] * v[0] + v[1] * v[1]
                sq23 = v[2] * v[2] + v[3] * v[3]
                sq45 = v[4] * v[4] + v[5] * v[5]
                sq67 = v[6] * v[6] + v[7] * v[7]
                sq = (sq01 + sq23) + (sq45 + sq67)
                sv = _bcast_last(plsc.cumsum(sq))
                ib = lax.bitcast_convert_type(sv, jnp.int32)
                y = lax.bitcast_convert_type(
                    jnp.int32(0x5F3759DF) - (ib >> 1), jnp.float32)
                h = sv * jnp.float32(-0.5)
                y = y * (jnp.float32(1.5) + h * y * y)
                y = y * (jnp.float32(1.5) + h * y * y)
                y = jnp.minimum(y, jnp.float32(1e12))
                nsqv = sv * y * y
                aux = jnp.where(lane == 0, nsqv,
                                jnp.where(lane == 1, jnp.float32(1.0),
                                          jnp.float32(0.0)))
                addr = lblv * jnp.int32(W)
                for j in range(8):
                    plsc.addupdate_scatter(tbl, [addr + col[j]], v[j] * y)
                plsc.addupdate_scatter(tbl, [addr + col[8]], aux)

            if g + 1 < NCHUNK:
                pending = nxt

        pltpu.sync_copy(tbl, out_hbm.at[wid])

    return body(xs, labels)


def _tc_partials(xs, labels3):
    def body(x_ref, lbl_ref, out_ref):
        g = pl.program_id(0)
        x = x_ref[...]
        nrm2 = jnp.sum(x * x, axis=1, keepdims=True)
        scale = jnp.minimum(lax.rsqrt(nrm2), jnp.float32(1e12))
        xh = x * scale
        nsq = nrm2 * scale * scale
        lbl = lbl_ref[0, 0]
        onehot_t = (lax.broadcasted_iota(jnp.int32, (CPAD, TC_BLK), 0) ==
                    lbl[None, :]).astype(jnp.float32)
        s_tbl = jax.lax.dot_general(
            onehot_t, xh, (((1,), (0,)), ((), ())),
            preferred_element_type=jnp.float32)
        a_tbl = jax.lax.dot_general(
            onehot_t, nsq, (((1,), (0,)), ((), ())),
            preferred_element_type=jnp.float32)
        cnt = jnp.sum(onehot_t, axis=1, keepdims=True)
        colidx = lax.broadcasted_iota(jnp.int32, (CPAD, W - D), 1)
        second = jnp.where(colidx == 0, a_tbl,
                           jnp.where(colidx == 1, cnt, jnp.float32(0.0)))
        part = jnp.concatenate([s_tbl, second], axis=1)

        @pl.when(g == 0)
        def _init():
            out_ref[...] = jnp.zeros((CPAD, W), jnp.float32)

        out_ref[...] += part

    return pl.pallas_call(
        body,
        grid=(NBLK - BLK0_TC,),
        in_specs=[
            pl.BlockSpec((TC_BLK, D), lambda g: (g + BLK0_TC, 0)),
            pl.BlockSpec((1, 1, TC_BLK), lambda g: (g + BLK0_TC, 0, 0)),
        ],
        out_specs=pl.BlockSpec((CPAD, W), lambda g: (0, 0)),
        out_shape=jax.ShapeDtypeStruct((CPAD, W), jnp.float32),
    )(xs, labels3)


def _combine(part_sc, part_tc, center):
    def body(psc_ref, ptc_ref, center_ref, out_ref):
        p3 = psc_ref[...].reshape(NW, CPAD, W)
        p = jnp.sum(p3, axis=0) + ptc_ref[...]
        c = center_ref[...]
        S = p[:CLS, :D]
        dot = jnp.sum(S * c, axis=1, keepdims=True)
        cnsq = jnp.sum(c * c, axis=1, keepdims=True)
        A = p[:CLS, D:D + 1]
        cnt = p[:CLS, D + 1:D + 2]
        per = jnp.where(cnt > 0,
                        (A - 2.0 * dot) / jnp.maximum(cnt, 1.0) + cnsq,
                        0.0)
        out_ref[...] = jnp.sum(per).reshape(1, 1)

    return pl.pallas_call(
        body,
        out_shape=jax.ShapeDtypeStruct((1, 1), jnp.float32),
    )(part_sc, part_tc, center)


def kernel(xs, label, center):
    labels = label.astype(jnp.int32)
    part_sc = _sc_partials(xs, labels)
    part_tc = _tc_partials(xs, labels.reshape(N // TC_BLK, 1, TC_BLK))
    out = _combine(part_sc, part_tc, center)
    return out[0, 0]
